# baseline (speedup 1.0000x reference)
import functools

import jax
import jax.numpy as jnp
from jax import lax
from jax.experimental import pallas as pl
from jax.experimental.pallas import tpu as pltpu
from jax.experimental.pallas import tpu_sc as plsc

N = 10000
F = 128
E = 320000
T = 12
L = 2

NPAD = 10240
CH = 128
EPW32 = 10240
NCH32 = EPW32 // CH
EPW16 = 20224
NCH16 = EPW16 // CH
ROWS_PER_TILE = NPAD // 16


def _zero_and_init_acc(rows0, acc_sh, s):
    zero16 = jnp.zeros((16,), jnp.float32)

    def _zrow(i, _):
        for j in range(8):
            rows0[i, pl.ds(j * 16, 16)] = zero16
        return 0

    lax.fori_loop(0, CH, _zrow, 0)
    base = s * ROWS_PER_TILE
    for b in range(ROWS_PER_TILE // CH):
        pltpu.sync_copy(rows0, acc_sh.at[pl.ds(base + b * CH, CH)])


def _seg_pipeline(tbl_hbm, sdx_hbm, w, nchunk, acc_sh, ib0, ib1,
                  rows0, rows1, gsem0, gsem1, ssem0, ssem1):
    pltpu.sync_copy(sdx_hbm.at[w, 0], ib0)
    pltpu.async_copy(tbl_hbm.at[ib0.at[0]], rows0, gsem0)
    pltpu.sync_copy(sdx_hbm.at[w, 1], ib1)
    pltpu.async_copy(tbl_hbm.at[ib1.at[0]], rows1, gsem1)

    def _body(i, _):
        c0 = 2 * i
        pltpu.make_async_copy(tbl_hbm.at[ib0.at[0]], rows0, gsem0).wait()
        pltpu.async_copy(rows0, acc_sh.at[ib0.at[1]], ssem0, add=True)
        pltpu.make_async_copy(tbl_hbm.at[ib1.at[0]], rows1, gsem1).wait()
        pltpu.async_copy(rows1, acc_sh.at[ib1.at[1]], ssem1, add=True)
        pltpu.make_async_copy(rows0, acc_sh.at[ib0.at[1]], ssem0).wait()
        pltpu.sync_copy(sdx_hbm.at[w, jnp.minimum(c0 + 2, nchunk - 1)], ib0)
        pltpu.async_copy(tbl_hbm.at[ib0.at[0]], rows0, gsem0)
        pltpu.make_async_copy(rows1, acc_sh.at[ib1.at[1]], ssem1).wait()
        pltpu.sync_copy(sdx_hbm.at[w, jnp.minimum(c0 + 3, nchunk - 1)], ib1)
        pltpu.async_copy(tbl_hbm.at[ib1.at[0]], rows1, gsem1)
        return 0

    lax.fori_loop(0, nchunk // 2, _body, 0)
    pltpu.make_async_copy(tbl_hbm.at[ib0.at[0]], rows0, gsem0).wait()
    pltpu.make_async_copy(tbl_hbm.at[ib1.at[0]], rows1, gsem1).wait()


def _sc_zr_body(sdx_hbm, pz_hbm, pr_hbm, out_hbm,
                ib0, ib1, rows0, rows1, acc_sh, gsem0, gsem1, ssem0, ssem1):
    c = lax.axis_index("c")
    s = lax.axis_index("s")
    _zero_and_init_acc(rows0, acc_sh, s)
    plsc.subcore_barrier()

    @pl.when(c == 0)
    def _():
        _seg_pipeline(pz_hbm, sdx_hbm, s, NCH16, acc_sh, ib0, ib1,
                      rows0, rows1, gsem0, gsem1, ssem0, ssem1)

    @pl.when(c == 1)
    def _():
        _seg_pipeline(pr_hbm, sdx_hbm, s, NCH16, acc_sh, ib0, ib1,
                      rows0, rows1, gsem0, gsem1, ssem0, ssem1)

    plsc.subcore_barrier()
    base = s * ROWS_PER_TILE
    pltpu.sync_copy(
        acc_sh.at[pl.ds(base, ROWS_PER_TILE)],
        out_hbm.at[c, pl.ds(base, ROWS_PER_TILE)],
    )


def _sc_partial_body(sdx_hbm, p_hbm, out_hbm,
                     ib0, ib1, rows0, rows1, acc_sh, gsem0, gsem1, ssem0, ssem1):
    c = lax.axis_index("c")
    s = lax.axis_index("s")
    w = s * 2 + c
    _zero_and_init_acc(rows0, acc_sh, s)
    plsc.subcore_barrier()
    _seg_pipeline(p_hbm, sdx_hbm, w, NCH32, acc_sh, ib0, ib1,
                  rows0, rows1, gsem0, gsem1, ssem0, ssem1)
    plsc.subcore_barrier()
    base = s * ROWS_PER_TILE
    pltpu.sync_copy(
        acc_sh.at[pl.ds(base, ROWS_PER_TILE)],
        out_hbm.at[c, pl.ds(base, ROWS_PER_TILE)],
    )


def _sc_scratch(nchunk):
    return [
        pltpu.VMEM((2, CH), jnp.int32),
        pltpu.VMEM((2, CH), jnp.int32),
        pltpu.VMEM((CH, F), jnp.float32),
        pltpu.VMEM((CH, F), jnp.float32),
        pltpu.VMEM_SHARED((NPAD, F), jnp.float32),
        pltpu.SemaphoreType.DMA,
        pltpu.SemaphoreType.DMA,
        pltpu.SemaphoreType.DMA,
        pltpu.SemaphoreType.DMA,
    ]


_sc_zr = pl.kernel(
    _sc_zr_body,
    out_type=jax.ShapeDtypeStruct((2, NPAD, F), jnp.float32),
    mesh=plsc.VectorSubcoreMesh(core_axis_name="c", subcore_axis_name="s"),
    scratch_types=_sc_scratch(NCH16),
)

_sc_partial = pl.kernel(
    _sc_partial_body,
    out_type=jax.ShapeDtypeStruct((2, NPAD, F), jnp.float32),
    mesh=plsc.VectorSubcoreMesh(core_axis_name="c", subcore_axis_name="s"),
    scratch_types=_sc_scratch(NCH32),
)


BN = 400
_GRID = N // BN


def _tck_prep_body(sd_ref, a_ref):
    a_ref[...] = lax.rsqrt(sd_ref[0] + sd_ref[1] + 1.0)


def _tck_a_body(x_ref, h_ref, a_ref, w_ref, pz_ref, pr_ref, xs_ref, hs_ref):
    a = a_ref[...]
    xs = a * x_ref[...]
    hs = a * h_ref[...]
    w = w_ref[...]
    acc = jnp.dot(xs, w[:F, :], preferred_element_type=jnp.float32)
    acc = acc + jnp.dot(hs, w[F:, :], preferred_element_type=jnp.float32)
    pz_ref[...] = acc[:, :F]
    pr_ref[...] = acc[:, F:]
    xs_ref[...] = xs
    hs_ref[...] = hs


def _tck_b_body(segzr_ref, pz_ref, pr_ref, a_ref, xs_ref, hs_ref,
                bzr_ref, wh_ref, ph_ref, z_ref):
    a = a_ref[...]
    bz = bzr_ref[0:1, :F]
    br = bzr_ref[0:1, F:]
    z = jax.nn.sigmoid(a * (segzr_ref[0] + pz_ref[...]) + bz)
    r = jax.nn.sigmoid(a * (segzr_ref[1] + pr_ref[...]) + br)
    rhs = r * hs_ref[...]
    wh = wh_ref[...]
    acc = jnp.dot(xs_ref[...], wh[:F, :], preferred_element_type=jnp.float32)
    acc = acc + jnp.dot(rhs, wh[F:, :], preferred_element_type=jnp.float32)
    ph_ref[...] = acc
    z_ref[...] = z


def _tck_c_body(segh_ref, ph_ref, a_ref, z_ref, h_ref, bh_ref, hn_ref):
    a = a_ref[...]
    ht = jnp.tanh(a * (segh_ref[0] + segh_ref[1] + ph_ref[...]) + bh_ref[0:1, :])
    z = z_ref[...]
    hn_ref[...] = z * h_ref[...] + (1.0 - z) * ht


def _row_spec(width):
    return pl.BlockSpec((BN, width), lambda i: (i, 0))


def _seg_spec():
    return pl.BlockSpec((2, BN, F), lambda i: (0, i, 0))


def _full_spec(r, c):
    return pl.BlockSpec((r, c), lambda i: (0, 0))


_tck_prep = pl.pallas_call(
    _tck_prep_body,
    grid=(_GRID,),
    in_specs=[_seg_spec()],
    out_specs=_row_spec(F),
    out_shape=jax.ShapeDtypeStruct((N, F), jnp.float32),
)

_tck_a = pl.pallas_call(
    _tck_a_body,
    grid=(_GRID,),
    in_specs=[_row_spec(F), _row_spec(F), _row_spec(F), _full_spec(2 * F, 2 * F)],
    out_specs=[_row_spec(F)] * 4,
    out_shape=[jax.ShapeDtypeStruct((N, F), jnp.float32)] * 4,
)

_tck_b = pl.pallas_call(
    _tck_b_body,
    grid=(_GRID,),
    in_specs=[_seg_spec(), _row_spec(F), _row_spec(F), _row_spec(F),
              _row_spec(F), _row_spec(F), _full_spec(8, 2 * F), _full_spec(2 * F, F)],
    out_specs=[_row_spec(F)] * 2,
    out_shape=[jax.ShapeDtypeStruct((N, F), jnp.float32)] * 2,
)

_tck_c = pl.pallas_call(
    _tck_c_body,
    grid=(_GRID,),
    in_specs=[_seg_spec(), _row_spec(F), _row_spec(F), _row_spec(F), _row_spec(F),
              _full_spec(8, F)],
    out_specs=_row_spec(F),
    out_shape=jax.ShapeDtypeStruct((N, F), jnp.float32),
)


def _pad_edges(src, dst, nworker, epw):
    pad = nworker * epw - E
    pad_src = (jnp.arange(pad, dtype=jnp.int32) * 37) % N
    pad_dst = N + (jnp.arange(pad, dtype=jnp.int32) % (NPAD - N))
    src_p = jnp.concatenate([src, pad_src]).reshape(nworker, epw // CH, 1, CH)
    dst_p = jnp.concatenate([dst, pad_dst]).reshape(nworker, epw // CH, 1, CH)
    return jnp.concatenate([src_p, dst_p], axis=2)


def kernel(input, edge_index, W_zr0, b_zr0, W_h0, b_h0, W_zr1, b_zr1, W_h1, b_h1):
    x_all = input[0]

    order = jnp.argsort(edge_index[0])
    src_s = edge_index[0][order]
    dst_s = edge_index[1][order]
    sdx16 = _pad_edges(src_s, dst_s, 16, EPW16)
    sdx32 = _pad_edges(src_s, dst_s, 32, EPW32)

    segdeg = _sc_partial(sdx32, jnp.ones((N, F), jnp.float32))
    a = _tck_prep(segdeg)

    bzr0b = jnp.broadcast_to(b_zr0[None, :], (8, 2 * F))
    bzr1b = jnp.broadcast_to(b_zr1[None, :], (8, 2 * F))
    bh0b = jnp.broadcast_to(b_h0[None, :], (8, F))
    bh1b = jnp.broadcast_to(b_h1[None, :], (8, F))
    params = [(W_zr0, bzr0b, W_h0, bh0b), (W_zr1, bzr1b, W_h1, bh1b)]

    hs_state = [jnp.zeros((N, F), jnp.float32) for _ in range(L)]
    outs = []
    for t in range(T):
        out = x_all[t]
        for l in range(L):
            Wzr, bzrb, Wh, bhb = params[l]
            h = hs_state[l]
            pz, pr, xs, hsc = _tck_a(out, h, a, Wzr)
            segzr = _sc_zr(sdx16, pz, pr)
            ph, z = _tck_b(segzr, pz, pr, a, xs, hsc, bzrb, Wh)
            segh = _sc_partial(sdx32, ph)
            hs_state[l] = _tck_c(segh, ph, a, z, h, bhb)
            out = hs_state[l]
        outs.append(out)

    output = jnp.stack(outs)[None]
    hidden_out = jnp.stack(hs_state)[None]
    return (output, hidden_out)


# speedup vs baseline: 2.3486x; 2.3486x over previous
import functools

import jax
import jax.numpy as jnp
from jax import lax
from jax.experimental import pallas as pl
from jax.experimental.pallas import tpu as pltpu
from jax.experimental.pallas import tpu_sc as plsc

N = 10000
F = 128
E = 320000
T = 12
L = 2

NPAD = 10112
CH = 128
EPW32 = 10368
NCH32 = EPW32 // CH
EPW16 = 20352
NCH16 = EPW16 // CH
ROWS_PER_TILE = NPAD // 16


def _zero_and_init_acc(rows0, acc_sh, s):
    zero16 = jnp.zeros((16,), jnp.float32)

    def _zrow(i, _):
        for j in range(8):
            rows0[i, pl.ds(j * 16, 16)] = zero16
        return 0

    lax.fori_loop(0, CH, _zrow, 0)
    base = s * ROWS_PER_TILE
    for b in range(ROWS_PER_TILE // CH):
        pltpu.sync_copy(rows0, acc_sh.at[pl.ds(base + b * CH, CH)])
    rem = ROWS_PER_TILE % CH
    if rem:
        pltpu.sync_copy(
            rows0.at[pl.ds(0, rem)],
            acc_sh.at[pl.ds(base + (ROWS_PER_TILE // CH) * CH, rem)],
        )


def _seg_pipeline(tbl_hbm, sdx_hbm, w, nchunk, acc_sh, ib0, ib1, ib2,
                  rows0, rows1, rows2, gsem0, gsem1, gsem2,
                  ssem0, ssem1, ssem2):
    bufs = ((ib0, rows0, gsem0, ssem0),
            (ib1, rows1, gsem1, ssem1),
            (ib2, rows2, gsem2, ssem2))

    for k, (ib, rows, gsem, _) in enumerate(bufs):
        pltpu.sync_copy(sdx_hbm.at[w, k], ib)
        pltpu.async_copy(tbl_hbm.at[ib.at[0]], rows, gsem)

    def _body(i, _):
        c0 = 3 * i
        for k, (ib, rows, gsem, ssem) in enumerate(bufs):
            pltpu.make_async_copy(tbl_hbm.at[ib.at[0]], rows, gsem).wait()
            pltpu.async_copy(rows, acc_sh.at[ib.at[1]], ssem, add=True)
            pltpu.make_async_copy(rows, acc_sh.at[ib.at[1]], ssem).wait()
            pltpu.sync_copy(
                sdx_hbm.at[w, jnp.minimum(c0 + k + 3, nchunk - 1)], ib)
            pltpu.async_copy(tbl_hbm.at[ib.at[0]], rows, gsem)
        return 0

    lax.fori_loop(0, nchunk // 3, _body, 0)
    for _, (ib, rows, gsem, _s) in enumerate(bufs):
        pltpu.make_async_copy(tbl_hbm.at[ib.at[0]], rows, gsem).wait()


def _sc_zr_body(sdx_hbm, pz_hbm, pr_hbm, out_hbm,
                ib0, ib1, ib2, rows0, rows1, rows2, acc_sh,
                gsem0, gsem1, gsem2, ssem0, ssem1, ssem2):
    c = lax.axis_index("c")
    s = lax.axis_index("s")
    _zero_and_init_acc(rows0, acc_sh, s)
    plsc.subcore_barrier()

    @pl.when(c == 0)
    def _():
        _seg_pipeline(pz_hbm, sdx_hbm, s, NCH16, acc_sh, ib0, ib1, ib2,
                      rows0, rows1, rows2, gsem0, gsem1, gsem2,
                      ssem0, ssem1, ssem2)

    @pl.when(c == 1)
    def _():
        _seg_pipeline(pr_hbm, sdx_hbm, s, NCH16, acc_sh, ib0, ib1, ib2,
                      rows0, rows1, rows2, gsem0, gsem1, gsem2,
                      ssem0, ssem1, ssem2)

    plsc.subcore_barrier()
    base = s * ROWS_PER_TILE
    pltpu.sync_copy(
        acc_sh.at[pl.ds(base, ROWS_PER_TILE)],
        out_hbm.at[c, pl.ds(base, ROWS_PER_TILE)],
    )


def _sc_partial_body(sdx_hbm, p_hbm, out_hbm,
                     ib0, ib1, ib2, rows0, rows1, rows2, acc_sh,
                     gsem0, gsem1, gsem2, ssem0, ssem1, ssem2):
    c = lax.axis_index("c")
    s = lax.axis_index("s")
    w = s * 2 + c
    _zero_and_init_acc(rows0, acc_sh, s)
    plsc.subcore_barrier()
    _seg_pipeline(p_hbm, sdx_hbm, w, NCH32, acc_sh, ib0, ib1, ib2,
                  rows0, rows1, rows2, gsem0, gsem1, gsem2,
                  ssem0, ssem1, ssem2)
    plsc.subcore_barrier()
    base = s * ROWS_PER_TILE
    pltpu.sync_copy(
        acc_sh.at[pl.ds(base, ROWS_PER_TILE)],
        out_hbm.at[c, pl.ds(base, ROWS_PER_TILE)],
    )


def _sc_scratch(nchunk):
    return [
        pltpu.VMEM((2, CH), jnp.int32),
        pltpu.VMEM((2, CH), jnp.int32),
        pltpu.VMEM((2, CH), jnp.int32),
        pltpu.VMEM((CH, F), jnp.float32),
        pltpu.VMEM((CH, F), jnp.float32),
        pltpu.VMEM((CH, F), jnp.float32),
        pltpu.VMEM_SHARED((NPAD, F), jnp.float32),
    ] + [pltpu.SemaphoreType.DMA] * 6


_sc_zr = pl.kernel(
    _sc_zr_body,
    out_type=jax.ShapeDtypeStruct((2, NPAD, F), jnp.float32),
    mesh=plsc.VectorSubcoreMesh(core_axis_name="c", subcore_axis_name="s"),
    scratch_types=_sc_scratch(NCH16),
)

_sc_partial = pl.kernel(
    _sc_partial_body,
    out_type=jax.ShapeDtypeStruct((2, NPAD, F), jnp.float32),
    mesh=plsc.VectorSubcoreMesh(core_axis_name="c", subcore_axis_name="s"),
    scratch_types=_sc_scratch(NCH32),
)


BN = 400
_GRID = N // BN


def _tck_prep_body(sd_ref, a_ref):
    a_ref[...] = lax.rsqrt(sd_ref[0] + sd_ref[1] + 1.0)


def _tck_a_body(x_ref, h_ref, a_ref, w_ref, pz_ref, pr_ref, xs_ref, hs_ref):
    a = a_ref[...]
    xs = a * x_ref[...]
    hs = a * h_ref[...]
    w = w_ref[...]
    acc = jnp.dot(xs, w[:F, :], preferred_element_type=jnp.float32)
    acc = acc + jnp.dot(hs, w[F:, :], preferred_element_type=jnp.float32)
    pz_ref[...] = acc[:, :F]
    pr_ref[...] = acc[:, F:]
    xs_ref[...] = xs
    hs_ref[...] = hs


def _tck_b_body(segzr_ref, pz_ref, pr_ref, a_ref, xs_ref, hs_ref,
                bzr_ref, wh_ref, ph_ref, z_ref):
    a = a_ref[...]
    bz = bzr_ref[0:1, :F]
    br = bzr_ref[0:1, F:]
    z = jax.nn.sigmoid(a * (segzr_ref[0] + pz_ref[...]) + bz)
    r = jax.nn.sigmoid(a * (segzr_ref[1] + pr_ref[...]) + br)
    rhs = r * hs_ref[...]
    wh = wh_ref[...]
    acc = jnp.dot(xs_ref[...], wh[:F, :], preferred_element_type=jnp.float32)
    acc = acc + jnp.dot(rhs, wh[F:, :], preferred_element_type=jnp.float32)
    ph_ref[...] = acc
    z_ref[...] = z


def _tck_c_body(segh_ref, ph_ref, a_ref, z_ref, h_ref, bh_ref, hn_ref):
    a = a_ref[...]
    ht = jnp.tanh(a * (segh_ref[0] + segh_ref[1] + ph_ref[...]) + bh_ref[0:1, :])
    z = z_ref[...]
    hn_ref[...] = z * h_ref[...] + (1.0 - z) * ht


def _row_spec(width):
    return pl.BlockSpec((BN, width), lambda i: (i, 0))


def _seg_spec():
    return pl.BlockSpec((2, BN, F), lambda i: (0, i, 0))


def _full_spec(r, c):
    return pl.BlockSpec((r, c), lambda i: (0, 0))


_tck_prep = pl.pallas_call(
    _tck_prep_body,
    grid=(_GRID,),
    in_specs=[_seg_spec()],
    out_specs=_row_spec(F),
    out_shape=jax.ShapeDtypeStruct((N, F), jnp.float32),
)

_tck_a = pl.pallas_call(
    _tck_a_body,
    grid=(_GRID,),
    in_specs=[_row_spec(F), _row_spec(F), _row_spec(F), _full_spec(2 * F, 2 * F)],
    out_specs=[_row_spec(F)] * 4,
    out_shape=[jax.ShapeDtypeStruct((N, F), jnp.float32)] * 4,
)

_tck_b = pl.pallas_call(
    _tck_b_body,
    grid=(_GRID,),
    in_specs=[_seg_spec(), _row_spec(F), _row_spec(F), _row_spec(F),
              _row_spec(F), _row_spec(F), _full_spec(8, 2 * F), _full_spec(2 * F, F)],
    out_specs=[_row_spec(F)] * 2,
    out_shape=[jax.ShapeDtypeStruct((N, F), jnp.float32)] * 2,
)

_tck_c = pl.pallas_call(
    _tck_c_body,
    grid=(_GRID,),
    in_specs=[_seg_spec(), _row_spec(F), _row_spec(F), _row_spec(F), _row_spec(F),
              _full_spec(8, F)],
    out_specs=_row_spec(F),
    out_shape=jax.ShapeDtypeStruct((N, F), jnp.float32),
)


def _pad_edges(src, dst, nworker, epw):
    pad = nworker * epw - E
    pad_src = (jnp.arange(pad, dtype=jnp.int32) * 37) % N
    pad_dst = N + (jnp.arange(pad, dtype=jnp.int32) % (NPAD - N))
    src_p = jnp.concatenate([src, pad_src]).reshape(nworker, epw // CH, 1, CH)
    dst_p = jnp.concatenate([dst, pad_dst]).reshape(nworker, epw // CH, 1, CH)
    return jnp.concatenate([src_p, dst_p], axis=2)


def kernel(input, edge_index, W_zr0, b_zr0, W_h0, b_h0, W_zr1, b_zr1, W_h1, b_h1):
    x_all = input[0]

    sdx16 = _pad_edges(edge_index[0], edge_index[1], 16, EPW16)
    sdx32 = _pad_edges(edge_index[0], edge_index[1], 32, EPW32)

    segdeg = _sc_partial(sdx32, jnp.ones((N, F), jnp.float32))
    a = _tck_prep(segdeg)

    bzr0b = jnp.broadcast_to(b_zr0[None, :], (8, 2 * F))
    bzr1b = jnp.broadcast_to(b_zr1[None, :], (8, 2 * F))
    bh0b = jnp.broadcast_to(b_h0[None, :], (8, F))
    bh1b = jnp.broadcast_to(b_h1[None, :], (8, F))
    params = [(W_zr0, bzr0b, W_h0, bh0b), (W_zr1, bzr1b, W_h1, bh1b)]

    hs_state = [jnp.zeros((N, F), jnp.float32) for _ in range(L)]
    outs = []
    for t in range(T):
        out = x_all[t]
        for l in range(L):
            Wzr, bzrb, Wh, bhb = params[l]
            h = hs_state[l]
            pz, pr, xs, hsc = _tck_a(out, h, a, Wzr)
            segzr = _sc_zr(sdx16, pz, pr)
            ph, z = _tck_b(segzr, pz, pr, a, xs, hsc, bzrb, Wh)
            segh = _sc_partial(sdx32, ph)
            hs_state[l] = _tck_c(segh, ph, a, z, h, bhb)
            out = hs_state[l]
        outs.append(out)

    output = jnp.stack(outs)[None]
    hidden_out = jnp.stack(hs_state)[None]
    return (output, hidden_out)
